# baseline (speedup 1.0000x reference)
import functools

import jax
import jax.numpy as jnp
from jax import lax
from jax.experimental import pallas as pl
from jax.experimental.pallas import tpu as pltpu
from jax.experimental.pallas import tpu_sc as plsc

NC = 2
NS = 16
NW = NC * NS


def _mesh():
    return plsc.VectorSubcoreMesh(
        core_axis_name="c", subcore_axis_name="s", num_cores=NC, num_subcores=NS
    )


_SC_PARAMS = pltpu.CompilerParams(
    use_tc_tiling_on_sc=False, needs_layout_passes=False
)


_Z16 = functools.partial(jnp.zeros, (16,), jnp.float32)


def _hist(dstf, nchunk, c, n_pad):
    per_tile = n_pad // NS
    kk = 5
    nblk = nchunk // kk

    @functools.partial(
        pl.kernel,
        out_type=jax.ShapeDtypeStruct((NC * n_pad,), jnp.float32),
        mesh=_mesh(),
        compiler_params=_SC_PARAMS,
        scratch_types=[
            pltpu.VMEM((2 * kk, c), jnp.int32),
            pltpu.VMEM((c,), jnp.float32),
            pltpu.VMEM((per_tile,), jnp.float32),
            pltpu.VMEM_SHARED((n_pad,), jnp.float32),
            pltpu.SemaphoreType.DMA,
        ],
    )
    def hist(dst_hbm, out_hbm, dst_ib, ones_v, zbuf_v, deg_sh, isem):
        cid = lax.axis_index("c")
        sid = lax.axis_index("s")
        wid = cid * NS + sid
        base = wid * nchunk
        for k in range(c // 16):
            ones_v[pl.ds(16 * k, 16)] = jnp.ones((16,), jnp.float32)
        for k in range(per_tile // 16):
            zbuf_v[pl.ds(16 * k, 16)] = _Z16()
        pltpu.sync_copy(zbuf_v, deg_sh.at[pl.ds(sid * per_tile, per_tile)])
        plsc.subcore_barrier()

        pltpu.sync_copy(dst_hbm.at[pl.ds(base, kk)], dst_ib.at[pl.ds(0, kk)])
        pltpu.async_copy(dst_hbm.at[pl.ds(base + kk, kk)], dst_ib.at[pl.ds(kk, kk)], isem)

        def body(t, carry):
            grp = lax.rem(t, 2) * kk
            for b in range(kk):
                pltpu.sync_copy(ones_v, deg_sh.at[dst_ib.at[grp + b]], add=True)

            @pl.when(t + 1 < nblk)
            def _():
                pltpu.make_async_copy(
                    dst_hbm.at[pl.ds(0, kk)], dst_ib.at[pl.ds(0, kk)], isem
                ).wait()

            @pl.when(t + 2 < nblk)
            def _():
                pltpu.async_copy(
                    dst_hbm.at[pl.ds(base + (t + 2) * kk, kk)],
                    dst_ib.at[pl.ds(grp, kk)], isem,
                )

            return carry

        lax.fori_loop(0, nblk, body, 0)
        plsc.subcore_barrier()
        pltpu.sync_copy(
            deg_sh.at[pl.ds(sid * per_tile, per_tile)],
            out_hbm.at[pl.ds(cid * n_pad + sid * per_tile, per_tile)],
        )

    return hist(dstf)


def _spmm(g_tab, srcf, dstf, nchunk, c, n, kk):
    _, fh = g_tab.shape
    wr = 1000
    owners = n // wr
    n_acc = n + 16
    zrows = 40
    nblk = nchunk // kk

    @functools.partial(
        pl.kernel,
        out_type=[
            jax.ShapeDtypeStruct((n, fh), jnp.float32),
            jax.ShapeDtypeStruct((n, fh), jnp.float32),
        ],
        mesh=_mesh(),
        compiler_params=_SC_PARAMS,
        scratch_types=[
            pltpu.VMEM((3 * kk, c), jnp.int32),
            pltpu.VMEM((3 * kk, c), jnp.int32),
            pltpu.VMEM((2 * kk * c, fh), jnp.float32),
            pltpu.VMEM((zrows, fh), jnp.float32),
            pltpu.VMEM_SHARED((n_acc, fh), jnp.float32),
            pltpu.SemaphoreType.DMA,
            pltpu.SemaphoreType.DMA,
            pltpu.SemaphoreType.DMA,
        ],
    )
    def spmm(g_hbm, src_hbm, dst_hbm, out0_hbm, out1_hbm, src_ib, dst_ib,
             rows_v, zbuf_v, acc_sh, gsem, ssem, isem):
        cid = lax.axis_index("c")
        sid = lax.axis_index("s")
        wid = cid * NS + sid
        base = wid * nchunk

        def zfill(i, carry):
            for k in range(fh // 16):
                zbuf_v[i, pl.ds(16 * k, 16)] = _Z16()
            return carry

        lax.fori_loop(0, zrows, zfill, 0)

        @pl.when(sid < owners)
        def _():
            def zcopy(k, carry):
                pltpu.sync_copy(zbuf_v, acc_sh.at[pl.ds(sid * wr + k * zrows, zrows)])
                return carry

            lax.fori_loop(0, wr // zrows, zcopy, 0)

        plsc.subcore_barrier()

        pltpu.sync_copy(src_hbm.at[pl.ds(base, kk)], src_ib.at[pl.ds(0, kk)])
        pltpu.sync_copy(dst_hbm.at[pl.ds(base, kk)], dst_ib.at[pl.ds(0, kk)])
        pltpu.async_copy(src_hbm.at[pl.ds(base + kk, kk)], src_ib.at[pl.ds(kk, kk)], isem)
        pltpu.async_copy(dst_hbm.at[pl.ds(base + kk, kk)], dst_ib.at[pl.ds(kk, kk)], isem)
        for b in range(kk):
            pltpu.async_copy(g_hbm.at[src_ib.at[b]], rows_v.at[pl.ds(b * c, c)], gsem)

        def block(t, carry):
            rg = lax.rem(t, 2) * kk
            g0 = lax.rem(t, 3) * kk
            g1 = lax.rem(t + 1, 3) * kk
            g2 = lax.rem(t + 2, 3) * kk
            for b in range(kk):
                pltpu.make_async_copy(
                    g_hbm.at[src_ib.at[0]], rows_v.at[pl.ds(0, c)], gsem
                ).wait()

            @pl.when(t >= 1)
            def _():
                for b in range(kk):
                    pltpu.make_async_copy(
                        g_hbm.at[src_ib.at[0]], rows_v.at[pl.ds(0, c)], ssem
                    ).wait()

            @pl.when(t + 2 < nblk)
            def _():
                pltpu.async_copy(
                    src_hbm.at[pl.ds(base + (t + 2) * kk, kk)],
                    src_ib.at[pl.ds(g2, kk)], isem,
                )
                pltpu.async_copy(
                    dst_hbm.at[pl.ds(base + (t + 2) * kk, kk)],
                    dst_ib.at[pl.ds(g2, kk)], isem,
                )

            for b in range(kk):
                pltpu.async_copy(
                    rows_v.at[pl.ds((rg + b) * c, c)],
                    acc_sh.at[dst_ib.at[g0 + b]], ssem, add=True,
                )

            @pl.when(t + 1 < nblk)
            def _():
                for b in range(2):
                    pltpu.make_async_copy(
                        src_hbm.at[pl.ds(0, kk)], src_ib.at[pl.ds(0, kk)], isem
                    ).wait()
                for b in range(kk):
                    pltpu.async_copy(
                        g_hbm.at[src_ib.at[g1 + b]],
                        rows_v.at[pl.ds((kk - rg + b) * c, c)], gsem,
                    )

            return carry

        lax.fori_loop(0, nblk, block, 0)
        for b in range(kk):
            pltpu.make_async_copy(
                g_hbm.at[src_ib.at[0]], rows_v.at[pl.ds(0, c)], ssem
            ).wait()
        plsc.subcore_barrier()

        @pl.when(sid < owners)
        def _():
            def wout(k, carry):
                r0 = sid * wr + k * 125
                sl = acc_sh.at[pl.ds(r0, 125)]

                @pl.when(cid == 0)
                def _():
                    pltpu.sync_copy(sl, out0_hbm.at[pl.ds(r0, 125)])

                @pl.when(cid == 1)
                def _():
                    pltpu.sync_copy(sl, out1_hbm.at[pl.ds(r0, 125)])

                return carry

            lax.fori_loop(0, wr // 125, wout, 0)

    return spmm(g_tab, srcf, dstf)


def _decode(z, a_r, b_r):
    n, f = z.shape
    nw, ncd, cd = a_r.shape

    @functools.partial(
        pl.kernel,
        out_type=jax.ShapeDtypeStruct((NW * ncd * cd,), jnp.float32),
        mesh=_mesh(),
        compiler_params=_SC_PARAMS,
        scratch_types=[
            pltpu.VMEM((ncd, cd), jnp.int32),
            pltpu.VMEM((ncd, cd), jnp.int32),
            pltpu.VMEM((2 * cd, f), jnp.float32),
            pltpu.VMEM((2 * cd, f), jnp.float32),
            pltpu.VMEM((cd,), jnp.float32),
            pltpu.SemaphoreType.DMA,
        ],
    )
    def decode(z_hbm, a_hbm, b_hbm, out_hbm, a_v, b_v, za_v, zb_v, lg_v, sem):
        cid = lax.axis_index("c")
        sid = lax.axis_index("s")
        wid = cid * NS + sid
        pltpu.sync_copy(a_hbm.at[wid], a_v)
        pltpu.sync_copy(b_hbm.at[wid], b_v)
        iota16 = lax.iota(jnp.int32, 16)
        pltpu.async_copy(z_hbm.at[a_v.at[0]], za_v.at[pl.ds(0, cd)], sem)
        pltpu.async_copy(z_hbm.at[b_v.at[0]], zb_v.at[pl.ds(0, cd)], sem)

        def chunk(j, carry):
            buf = lax.rem(j, 2) * cd
            for _ in range(2):
                pltpu.make_async_copy(
                    z_hbm.at[a_v.at[0]], za_v.at[pl.ds(0, cd)], sem
                ).wait()

            @pl.when(j + 1 < ncd)
            def _():
                nbuf = cd - buf
                pltpu.async_copy(z_hbm.at[a_v.at[j + 1]], za_v.at[pl.ds(nbuf, cd)], sem)
                pltpu.async_copy(z_hbm.at[b_v.at[j + 1]], zb_v.at[pl.ds(nbuf, cd)], sem)

            for grp in range(cd // 16):
                rows = iota16 + (16 * grp) + buf

                def col8(t, acc):
                    base = jnp.full((16,), 8 * t, jnp.int32)
                    for k in range(8):
                        cols = base + k
                        acc = acc + plsc.load_gather(
                            za_v, [rows, cols]
                        ) * plsc.load_gather(zb_v, [rows, cols])
                    return acc

                lg_v[pl.ds(16 * grp, 16)] = lax.fori_loop(0, f // 8, col8, _Z16())
            pltpu.sync_copy(lg_v, out_hbm.at[pl.ds(wid * ncd * cd + j * cd, cd)])
            return carry

        lax.fori_loop(0, ncd, chunk, 0)

    return decode(z, a_r, b_r)


def _tc1(x, w1stk, d0, d1):
    n, k = x.shape
    hh = w1stk.shape[2]
    r = 2000
    gi = n // r

    def body(x_ref, w_ref, d0_ref, d1_ref, g_ref, dinv_ref):
        deg = 1.0 + d0_ref[...] + d1_ref[...]
        dinv = lax.rsqrt(deg)
        hm = jnp.dot(x_ref[...], w_ref[0], preferred_element_type=jnp.float32)
        g_ref[...] = dinv * hm
        dinv_ref[...] = dinv

    return pl.pallas_call(
        body,
        grid=(gi, 2),
        in_specs=[
            pl.BlockSpec((r, k), lambda i, j: (i, 0)),
            pl.BlockSpec((1, k, hh), lambda i, j: (j, 0, 0)),
            pl.BlockSpec((r, 1), lambda i, j: (i, 0)),
            pl.BlockSpec((r, 1), lambda i, j: (i, 0)),
        ],
        out_specs=[
            pl.BlockSpec((r, hh), lambda i, j: (j * gi + i, 0)),
            pl.BlockSpec((r, 1), lambda i, j: (i, 0)),
        ],
        out_shape=[
            jax.ShapeDtypeStruct((2 * n, hh), jnp.float32),
            jax.ShapeDtypeStruct((n, 1), jnp.float32),
        ],
    )(x, w1stk, d0, d1)


def _tc2(g1s, s0, s1, dinv, b1, w2stk):
    n, hh = s0.shape
    h = 2 * hh
    dh = w2stk.shape[2]
    r = 2000
    gi = n // r

    def body(gl_ref, gr_ref, s0_ref, s1_ref, di_ref, b_ref, w_ref, o_ref):
        di = di_ref[...]
        ul = jnp.maximum(di * (s0_ref[...] + gl_ref[...]) + b_ref[:, :hh], 0.0)
        ur = jnp.maximum(di * (s1_ref[...] + gr_ref[...]) + b_ref[:, hh:], 0.0)
        u = jnp.concatenate([ul, ur], axis=1)
        o_ref[...] = di * jnp.dot(u, w_ref[0], preferred_element_type=jnp.float32)

    return pl.pallas_call(
        body,
        grid=(gi, 2),
        in_specs=[
            pl.BlockSpec((r, hh), lambda i, j: (i, 0)),
            pl.BlockSpec((r, hh), lambda i, j: (gi + i, 0)),
            pl.BlockSpec((r, hh), lambda i, j: (i, 0)),
            pl.BlockSpec((r, hh), lambda i, j: (i, 0)),
            pl.BlockSpec((r, 1), lambda i, j: (i, 0)),
            pl.BlockSpec((1, h), lambda i, j: (0, 0)),
            pl.BlockSpec((1, h, dh), lambda i, j: (j, 0, 0)),
        ],
        out_specs=pl.BlockSpec((r, dh), lambda i, j: (j * gi + i, 0)),
        out_shape=jax.ShapeDtypeStruct((2 * n, dh), jnp.float32),
    )(g1s, g1s, s0, s1, dinv, b1, w2stk)


def _tc3(g2s, s0, s1, dinv, b2):
    n2, dh = g2s.shape
    n = n2 // 2
    dout = 2 * dh
    r = 2000
    gi = n // r

    def body(gl_ref, gr_ref, s0_ref, s1_ref, di_ref, b_ref, o_ref):
        di = di_ref[...]
        zl = di * (s0_ref[...] + gl_ref[...]) + b_ref[:, :dh]
        zr = di * (s1_ref[...] + gr_ref[...]) + b_ref[:, dh:]
        o_ref[...] = jnp.concatenate([zl, zr], axis=1)

    return pl.pallas_call(
        body,
        grid=(gi,),
        in_specs=[
            pl.BlockSpec((r, dh), lambda i: (i, 0)),
            pl.BlockSpec((r, dh), lambda i: (gi + i, 0)),
            pl.BlockSpec((r, dh), lambda i: (i, 0)),
            pl.BlockSpec((r, dh), lambda i: (i, 0)),
            pl.BlockSpec((r, 1), lambda i: (i, 0)),
            pl.BlockSpec((1, dout), lambda i: (0, 0)),
        ],
        out_specs=pl.BlockSpec((r, dout), lambda i: (i, 0)),
        out_shape=jax.ShapeDtypeStruct((n, dout), jnp.float32),
    )(g2s, g2s, s0, s1, dinv, b2)


def kernel(x, edge_index, edge_label_index, W1, b1, W2, b2):
    n, _ = x.shape
    e = edge_index.shape[1]
    l = edge_label_index.shape[1]

    c = 128
    dummy = n + 8
    n_pad = 640 * NS

    ept_h = e // NW
    nch_h = -(-ept_h // c)
    nch_h += (-nch_h) % 5
    dsth = jnp.pad(
        edge_index[1].reshape(NW, ept_h),
        ((0, 0), (0, nch_h * c - ept_h)), constant_values=n + 100,
    ).reshape(NW * nch_h, c)
    degp = _hist(dsth, nch_h, c, n_pad).reshape(NC, n_pad)
    d0 = degp[0, :n, None]
    d1 = degp[1, :n, None]

    ept = e // NS
    nch = -(-ept // c)
    nch += (-nch) % 4
    src16 = jnp.pad(
        edge_index[0].reshape(NS, ept), ((0, 0), (0, nch * c - ept))
    ).reshape(NS, nch, c)
    srcf = jnp.concatenate([src16, src16 + n], axis=0).reshape(NW * nch, c)
    dst16 = jnp.pad(
        edge_index[1].reshape(NS, ept),
        ((0, 0), (0, nch * c - ept)), constant_values=dummy,
    ).reshape(NS, nch, c)
    dstf = jnp.concatenate([dst16, dst16], axis=0).reshape(NW * nch, c)

    hh = W1.shape[1] // 2
    dh = W2.shape[1] // 2
    w1stk = jnp.stack([W1[:, :hh], W1[:, hh:]])
    w2stk = jnp.stack([W2[:, :dh], W2[:, dh:]])

    g1s, dinv = _tc1(x, w1stk, d0, d1)
    s1l, s1r = _spmm(g1s, srcf, dstf, nch, c, n, 4)
    g2s = _tc2(g1s, s1l, s1r, dinv, b1.reshape(1, -1), w2stk)
    s2l, s2r = _spmm(g2s, srcf, dstf, nch, c, n, 4)
    z = _tc3(g2s, s2l, s2r, dinv, b2.reshape(1, -1))

    cd = 64
    ppw = l // NW
    ncd = 10
    eli = edge_label_index.reshape(2, NW, ppw)
    eli = jnp.pad(eli, ((0, 0), (0, 0), (0, ncd * cd - ppw)))
    a_r = eli[0].reshape(NW, ncd, cd)
    b_r = eli[1].reshape(NW, ncd, cd)
    lp = _decode(z, a_r, b_r)
    return lp.reshape(NW, ncd * cd)[:, :ppw].reshape(l)


# speedup vs baseline: 1.5214x; 1.5214x over previous
import functools

import jax
import jax.numpy as jnp
from jax import lax
from jax.experimental import pallas as pl
from jax.experimental.pallas import tpu as pltpu
from jax.experimental.pallas import tpu_sc as plsc

NC = 2
NS = 16
NW = NC * NS


def _mesh():
    return plsc.VectorSubcoreMesh(
        core_axis_name="c", subcore_axis_name="s", num_cores=NC, num_subcores=NS
    )


_SC_PARAMS = pltpu.CompilerParams(
    use_tc_tiling_on_sc=False, needs_layout_passes=False
)


_Z16 = functools.partial(jnp.zeros, (16,), jnp.float32)


def _hist(dstf, nchunk, c, n_pad):
    per_tile = n_pad // NS
    kk = 5
    nblk = nchunk // kk

    @functools.partial(
        pl.kernel,
        out_type=jax.ShapeDtypeStruct((NC * n_pad,), jnp.float32),
        mesh=_mesh(),
        compiler_params=_SC_PARAMS,
        scratch_types=[
            pltpu.VMEM((2 * kk, c), jnp.int32),
            pltpu.VMEM((c,), jnp.float32),
            pltpu.VMEM((per_tile,), jnp.float32),
            pltpu.VMEM_SHARED((n_pad,), jnp.float32),
            pltpu.SemaphoreType.DMA,
        ],
    )
    def hist(dst_hbm, out_hbm, dst_ib, ones_v, zbuf_v, deg_sh, isem):
        cid = lax.axis_index("c")
        sid = lax.axis_index("s")
        wid = cid * NS + sid
        base = wid * nchunk
        for k in range(c // 16):
            ones_v[pl.ds(16 * k, 16)] = jnp.ones((16,), jnp.float32)
        for k in range(per_tile // 16):
            zbuf_v[pl.ds(16 * k, 16)] = _Z16()
        pltpu.sync_copy(zbuf_v, deg_sh.at[pl.ds(sid * per_tile, per_tile)])
        plsc.subcore_barrier()

        pltpu.sync_copy(dst_hbm.at[pl.ds(base, kk)], dst_ib.at[pl.ds(0, kk)])
        pltpu.async_copy(dst_hbm.at[pl.ds(base + kk, kk)], dst_ib.at[pl.ds(kk, kk)], isem)

        def body(t, carry):
            grp = lax.rem(t, 2) * kk
            for b in range(kk):
                pltpu.sync_copy(ones_v, deg_sh.at[dst_ib.at[grp + b]], add=True)

            @pl.when(t + 1 < nblk)
            def _():
                pltpu.make_async_copy(
                    dst_hbm.at[pl.ds(0, kk)], dst_ib.at[pl.ds(0, kk)], isem
                ).wait()

            @pl.when(t + 2 < nblk)
            def _():
                pltpu.async_copy(
                    dst_hbm.at[pl.ds(base + (t + 2) * kk, kk)],
                    dst_ib.at[pl.ds(grp, kk)], isem,
                )

            return carry

        lax.fori_loop(0, nblk, body, 0)
        plsc.subcore_barrier()
        pltpu.sync_copy(
            deg_sh.at[pl.ds(sid * per_tile, per_tile)],
            out_hbm.at[pl.ds(cid * n_pad + sid * per_tile, per_tile)],
        )

    return hist(dstf)


def _spmm(g_tab, srcf, dstf, nchunk, c, n, kk):
    _, fh = g_tab.shape
    wr = 1000
    owners = n // wr
    n_acc = n + 16
    zrows = 40
    nblk = nchunk // kk

    @functools.partial(
        pl.kernel,
        out_type=[
            jax.ShapeDtypeStruct((n, fh), jnp.float32),
            jax.ShapeDtypeStruct((n, fh), jnp.float32),
        ],
        mesh=_mesh(),
        compiler_params=_SC_PARAMS,
        scratch_types=[
            pltpu.VMEM((3 * kk, c), jnp.int32),
            pltpu.VMEM((3 * kk, c), jnp.int32),
            pltpu.VMEM((2 * kk * c, fh), jnp.float32),
            pltpu.VMEM((zrows, fh), jnp.float32),
            pltpu.VMEM_SHARED((n_acc, fh), jnp.float32),
            pltpu.SemaphoreType.DMA,
            pltpu.SemaphoreType.DMA,
            pltpu.SemaphoreType.DMA,
        ],
    )
    def spmm(g_hbm, src_hbm, dst_hbm, out0_hbm, out1_hbm, src_ib, dst_ib,
             rows_v, zbuf_v, acc_sh, gsem, ssem, isem):
        cid = lax.axis_index("c")
        sid = lax.axis_index("s")
        wid = cid * NS + sid
        base = wid * nchunk

        def zfill(i, carry):
            for k in range(fh // 16):
                zbuf_v[i, pl.ds(16 * k, 16)] = _Z16()
            return carry

        lax.fori_loop(0, zrows, zfill, 0)

        @pl.when(sid < owners)
        def _():
            def zcopy(k, carry):
                pltpu.sync_copy(zbuf_v, acc_sh.at[pl.ds(sid * wr + k * zrows, zrows)])
                return carry

            lax.fori_loop(0, wr // zrows, zcopy, 0)

        plsc.subcore_barrier()

        pltpu.sync_copy(src_hbm.at[pl.ds(base, kk)], src_ib.at[pl.ds(0, kk)])
        pltpu.sync_copy(dst_hbm.at[pl.ds(base, kk)], dst_ib.at[pl.ds(0, kk)])
        pltpu.async_copy(src_hbm.at[pl.ds(base + kk, kk)], src_ib.at[pl.ds(kk, kk)], isem)
        pltpu.async_copy(dst_hbm.at[pl.ds(base + kk, kk)], dst_ib.at[pl.ds(kk, kk)], isem)
        for b in range(kk):
            pltpu.async_copy(g_hbm.at[src_ib.at[b]], rows_v.at[pl.ds(b * c, c)], gsem)

        def block(t, carry):
            rg = lax.rem(t, 2) * kk
            g0 = lax.rem(t, 3) * kk
            g1 = lax.rem(t + 1, 3) * kk
            g2 = lax.rem(t + 2, 3) * kk
            for b in range(kk):
                pltpu.make_async_copy(
                    g_hbm.at[src_ib.at[0]], rows_v.at[pl.ds(0, c)], gsem
                ).wait()

            @pl.when(t >= 1)
            def _():
                for b in range(kk):
                    pltpu.make_async_copy(
                        g_hbm.at[src_ib.at[0]], rows_v.at[pl.ds(0, c)], ssem
                    ).wait()

            @pl.when(t + 2 < nblk)
            def _():
                pltpu.async_copy(
                    src_hbm.at[pl.ds(base + (t + 2) * kk, kk)],
                    src_ib.at[pl.ds(g2, kk)], isem,
                )
                pltpu.async_copy(
                    dst_hbm.at[pl.ds(base + (t + 2) * kk, kk)],
                    dst_ib.at[pl.ds(g2, kk)], isem,
                )

            for b in range(kk):
                pltpu.async_copy(
                    rows_v.at[pl.ds((rg + b) * c, c)],
                    acc_sh.at[dst_ib.at[g0 + b]], ssem, add=True,
                )

            @pl.when(t + 1 < nblk)
            def _():
                for b in range(2):
                    pltpu.make_async_copy(
                        src_hbm.at[pl.ds(0, kk)], src_ib.at[pl.ds(0, kk)], isem
                    ).wait()
                for b in range(kk):
                    pltpu.async_copy(
                        g_hbm.at[src_ib.at[g1 + b]],
                        rows_v.at[pl.ds((kk - rg + b) * c, c)], gsem,
                    )

            return carry

        lax.fori_loop(0, nblk, block, 0)
        for b in range(kk):
            pltpu.make_async_copy(
                g_hbm.at[src_ib.at[0]], rows_v.at[pl.ds(0, c)], ssem
            ).wait()
        plsc.subcore_barrier()

        @pl.when(sid < owners)
        def _():
            def wout(k, carry):
                r0 = sid * wr + k * 125
                sl = acc_sh.at[pl.ds(r0, 125)]

                @pl.when(cid == 0)
                def _():
                    pltpu.sync_copy(sl, out0_hbm.at[pl.ds(r0, 125)])

                @pl.when(cid == 1)
                def _():
                    pltpu.sync_copy(sl, out1_hbm.at[pl.ds(r0, 125)])

                return carry

            lax.fori_loop(0, wr // 125, wout, 0)

    return spmm(g_tab, srcf, dstf)


def _decode(z, a_r, b_r):
    n, f = z.shape
    nw, ncd, cd = a_r.shape

    @functools.partial(
        pl.kernel,
        out_type=jax.ShapeDtypeStruct((NW * ncd * cd,), jnp.float32),
        mesh=_mesh(),
        compiler_params=_SC_PARAMS,
        scratch_types=[
            pltpu.VMEM((ncd, cd), jnp.int32),
            pltpu.VMEM((ncd, cd), jnp.int32),
            pltpu.VMEM((2 * cd, f), jnp.float32),
            pltpu.VMEM((2 * cd, f), jnp.float32),
            pltpu.VMEM((cd,), jnp.float32),
            pltpu.SemaphoreType.DMA,
        ],
    )
    def decode(z_hbm, a_hbm, b_hbm, out_hbm, a_v, b_v, za_v, zb_v, lg_v, sem):
        cid = lax.axis_index("c")
        sid = lax.axis_index("s")
        wid = cid * NS + sid
        pltpu.sync_copy(a_hbm.at[wid], a_v)
        pltpu.sync_copy(b_hbm.at[wid], b_v)
        iota16 = lax.iota(jnp.int32, 16)
        pltpu.async_copy(z_hbm.at[a_v.at[0]], za_v.at[pl.ds(0, cd)], sem)
        pltpu.async_copy(z_hbm.at[b_v.at[0]], zb_v.at[pl.ds(0, cd)], sem)

        def chunk(j, carry):
            buf = lax.rem(j, 2) * cd
            for _ in range(2):
                pltpu.make_async_copy(
                    z_hbm.at[a_v.at[0]], za_v.at[pl.ds(0, cd)], sem
                ).wait()

            @pl.when(j + 1 < ncd)
            def _():
                nbuf = cd - buf
                pltpu.async_copy(z_hbm.at[a_v.at[j + 1]], za_v.at[pl.ds(nbuf, cd)], sem)
                pltpu.async_copy(z_hbm.at[b_v.at[j + 1]], zb_v.at[pl.ds(nbuf, cd)], sem)

            for grp in range(cd // 16):
                rows = iota16 + (16 * grp) + buf

                def col8(t, acc):
                    base = jnp.full((16,), 8 * t, jnp.int32)
                    for k in range(8):
                        cols = base + k
                        acc = acc + plsc.load_gather(
                            za_v, [rows, cols]
                        ) * plsc.load_gather(zb_v, [rows, cols])
                    return acc

                lg_v[pl.ds(16 * grp, 16)] = lax.fori_loop(0, f // 8, col8, _Z16())
            pltpu.sync_copy(lg_v, out_hbm.at[pl.ds(wid * ncd * cd + j * cd, cd)])
            return carry

        lax.fori_loop(0, ncd, chunk, 0)

    return decode(z, a_r, b_r)


def _tc1(x, w1stk, d0, d1):
    n, k = x.shape
    hh = w1stk.shape[2]
    r = 2000
    gi = n // r

    def body(x_ref, w_ref, d0_ref, d1_ref, g_ref, dinv_ref):
        deg = 1.0 + d0_ref[...] + d1_ref[...]
        dinv = lax.rsqrt(deg)
        hm = jnp.dot(x_ref[...], w_ref[0], preferred_element_type=jnp.float32)
        g_ref[...] = dinv * hm
        dinv_ref[...] = dinv

    return pl.pallas_call(
        body,
        grid=(gi, 2),
        in_specs=[
            pl.BlockSpec((r, k), lambda i, j: (i, 0)),
            pl.BlockSpec((1, k, hh), lambda i, j: (j, 0, 0)),
            pl.BlockSpec((r, 1), lambda i, j: (i, 0)),
            pl.BlockSpec((r, 1), lambda i, j: (i, 0)),
        ],
        out_specs=[
            pl.BlockSpec((r, hh), lambda i, j: (j * gi + i, 0)),
            pl.BlockSpec((r, 1), lambda i, j: (i, 0)),
        ],
        out_shape=[
            jax.ShapeDtypeStruct((2 * n, hh), jnp.float32),
            jax.ShapeDtypeStruct((n, 1), jnp.float32),
        ],
    )(x, w1stk, d0, d1)


def _tc2(g1s, s0, s1, dinv, b1, w2stk):
    n, hh = s0.shape
    h = 2 * hh
    dh = w2stk.shape[2]
    r = 2000
    gi = n // r

    def body(gl_ref, gr_ref, s0_ref, s1_ref, di_ref, b_ref, w_ref, o_ref):
        di = di_ref[...]
        ul = jnp.maximum(di * (s0_ref[...] + gl_ref[...]) + b_ref[:, :hh], 0.0)
        ur = jnp.maximum(di * (s1_ref[...] + gr_ref[...]) + b_ref[:, hh:], 0.0)
        u = jnp.concatenate([ul, ur], axis=1)
        o_ref[...] = di * jnp.dot(u, w_ref[0], preferred_element_type=jnp.float32)

    return pl.pallas_call(
        body,
        grid=(gi, 2),
        in_specs=[
            pl.BlockSpec((r, hh), lambda i, j: (i, 0)),
            pl.BlockSpec((r, hh), lambda i, j: (gi + i, 0)),
            pl.BlockSpec((r, hh), lambda i, j: (i, 0)),
            pl.BlockSpec((r, hh), lambda i, j: (i, 0)),
            pl.BlockSpec((r, 1), lambda i, j: (i, 0)),
            pl.BlockSpec((1, h), lambda i, j: (0, 0)),
            pl.BlockSpec((1, h, dh), lambda i, j: (j, 0, 0)),
        ],
        out_specs=pl.BlockSpec((r, dh), lambda i, j: (j * gi + i, 0)),
        out_shape=jax.ShapeDtypeStruct((2 * n, dh), jnp.float32),
    )(g1s, g1s, s0, s1, dinv, b1, w2stk)


def _tc3(g2s, s0, s1, dinv, b2):
    n2, dh = g2s.shape
    n = n2 // 2
    dout = 2 * dh
    r = 2000
    gi = n // r

    def body(gl_ref, gr_ref, s0_ref, s1_ref, di_ref, b_ref, o_ref):
        di = di_ref[...]
        zl = di * (s0_ref[...] + gl_ref[...]) + b_ref[:, :dh]
        zr = di * (s1_ref[...] + gr_ref[...]) + b_ref[:, dh:]
        o_ref[...] = jnp.concatenate([zl, zr], axis=1)

    return pl.pallas_call(
        body,
        grid=(gi,),
        in_specs=[
            pl.BlockSpec((r, dh), lambda i: (i, 0)),
            pl.BlockSpec((r, dh), lambda i: (gi + i, 0)),
            pl.BlockSpec((r, dh), lambda i: (i, 0)),
            pl.BlockSpec((r, dh), lambda i: (i, 0)),
            pl.BlockSpec((r, 1), lambda i: (i, 0)),
            pl.BlockSpec((1, dout), lambda i: (0, 0)),
        ],
        out_specs=pl.BlockSpec((r, dout), lambda i: (i, 0)),
        out_shape=jax.ShapeDtypeStruct((n, dout), jnp.float32),
    )(g2s, g2s, s0, s1, dinv, b2)


def kernel(x, edge_index, edge_label_index, W1, b1, W2, b2):
    n, _ = x.shape
    e = edge_index.shape[1]
    l = edge_label_index.shape[1]

    c = 80
    n_pad = 640 * NS

    ept_h = e // NW
    nch_h = ept_h // c
    dsth = edge_index[1].reshape(NW * nch_h, c)
    degp = _hist(dsth, nch_h, c, n_pad).reshape(NC, n_pad)
    d0 = degp[0, :n, None]
    d1 = degp[1, :n, None]

    ept = e // NS
    nch = ept // c
    src16 = edge_index[0].reshape(NS, nch, c)
    srcf = jnp.concatenate([src16, src16 + n], axis=0).reshape(NW * nch, c)
    dst16 = edge_index[1].reshape(NS, nch, c)
    dstf = jnp.concatenate([dst16, dst16], axis=0).reshape(NW * nch, c)

    hh = W1.shape[1] // 2
    dh = W2.shape[1] // 2
    w1stk = jnp.stack([W1[:, :hh], W1[:, hh:]])
    w2stk = jnp.stack([W2[:, :dh], W2[:, dh:]])

    g1s, dinv = _tc1(x, w1stk, d0, d1)
    s1l, s1r = _spmm(g1s, srcf, dstf, nch, c, n, 5)
    g2s = _tc2(g1s, s1l, s1r, dinv, b1.reshape(1, -1), w2stk)
    s2l, s2r = _spmm(g2s, srcf, dstf, nch, c, n, 5)
    z = _tc3(g2s, s2l, s2r, dinv, b2.reshape(1, -1))

    cd = 64
    ppw = l // NW
    ncd = 10
    eli = edge_label_index.reshape(2, NW, ppw)
    eli = jnp.pad(eli, ((0, 0), (0, 0), (0, ncd * cd - ppw)))
    a_r = eli[0].reshape(NW, ncd, cd)
    b_r = eli[1].reshape(NW, ncd, cd)
    lp = _decode(z, a_r, b_r)
    return lp.reshape(NW, ncd * cd)[:, :ppw].reshape(l)
